# baseline (speedup 1.0000x reference)
import jax
import jax.numpy as jnp
from jax.experimental import pallas as pl

_B = 16
_HW = {8: 64 * 64, 16: 32 * 32, 32: 16 * 16}
_THR = 0.5


def _body(t0c8, t1c8, t0b8, t0t8, t1b8, t1t8,
          t0c16, t1c16, t0b16, t0t16, t1b16, t1t16,
          t0c32, t0b32, t0t32,
          o_t0b8, o_t0t8, o_t1b8, o_t1t8,
          o_t0b16, o_t0t16, o_t1b16, o_t1t16,
          o_t0b32, o_t0t32):
    def one_stride(c0, c1, pairs):
        mx = jnp.max(c0[0], axis=0)
        if c1 is not None:
            mx = jnp.maximum(mx, jnp.max(c1[0], axis=0))
        m = (mx > _THR).astype(jnp.float32)[None, :]
        for i_ref, o_ref in pairs:
            o_ref[0] = i_ref[0] * m

    one_stride(t0c8, t1c8,
               [(t0b8, o_t0b8), (t0t8, o_t0t8), (t1b8, o_t1b8), (t1t8, o_t1t8)])
    one_stride(t0c16, t1c16,
               [(t0b16, o_t0b16), (t0t16, o_t0t16), (t1b16, o_t1b16), (t1t16, o_t1t16)])
    one_stride(t0c32, None,
               [(t0b32, o_t0b32), (t0t32, o_t0t32)])


def _spec(c, hw):
    return pl.BlockSpec((1, c, hw), lambda n: (n, 0, 0))


def kernel(t0_cls_s8, t0_cls_s16, t0_cls_s32,
           t0_box_s8, t0_box_s16, t0_box_s32,
           t0_ctr_s8, t0_ctr_s16, t0_ctr_s32,
           t1_cls_s8, t1_cls_s16,
           t1_box_s8, t1_box_s16,
           t1_ctr_s8, t1_ctr_s16):
    def flat(x):
        n, c, h, w = x.shape
        return x.reshape(n, c, h * w)

    ins = [flat(t0_cls_s8), flat(t1_cls_s8),
           flat(t0_box_s8), flat(t0_ctr_s8), flat(t1_box_s8), flat(t1_ctr_s8),
           flat(t0_cls_s16), flat(t1_cls_s16),
           flat(t0_box_s16), flat(t0_ctr_s16), flat(t1_box_s16), flat(t1_ctr_s16),
           flat(t0_cls_s32), flat(t0_box_s32), flat(t0_ctr_s32)]

    in_specs = [_spec(x.shape[1], x.shape[2]) for x in ins]
    out_shapes = [
        jax.ShapeDtypeStruct((_B, 4, _HW[8]), jnp.float32),
        jax.ShapeDtypeStruct((_B, 1, _HW[8]), jnp.float32),
        jax.ShapeDtypeStruct((_B, 4, _HW[8]), jnp.float32),
        jax.ShapeDtypeStruct((_B, 1, _HW[8]), jnp.float32),
        jax.ShapeDtypeStruct((_B, 4, _HW[16]), jnp.float32),
        jax.ShapeDtypeStruct((_B, 1, _HW[16]), jnp.float32),
        jax.ShapeDtypeStruct((_B, 4, _HW[16]), jnp.float32),
        jax.ShapeDtypeStruct((_B, 1, _HW[16]), jnp.float32),
        jax.ShapeDtypeStruct((_B, 4, _HW[32]), jnp.float32),
        jax.ShapeDtypeStruct((_B, 1, _HW[32]), jnp.float32),
    ]
    out_specs = [_spec(s.shape[1], s.shape[2]) for s in out_shapes]

    outs = pl.pallas_call(
        _body,
        grid=(_B,),
        in_specs=in_specs,
        out_specs=out_specs,
        out_shape=out_shapes,
    )(*ins)

    dims = {8: (64, 64), 16: (32, 32), 32: (16, 16)}
    (b8_0, c8_0, b8_1, c8_1,
     b16_0, c16_0, b16_1, c16_1,
     b32_0, c32_0) = [o.reshape(o.shape[0], o.shape[1], *dims[s])
                      for o, s in zip(outs, [8] * 4 + [16] * 4 + [32] * 2)]

    result = []
    for group in ([b8_0, c8_0, b8_1, c8_1],
                  [b16_0, c16_0, b16_1, c16_1],
                  [b32_0, c32_0]):
        for n in range(_B):
            for d in group:
                result.append(d[n])
    return tuple(result)
